# baseline (speedup 1.0000x reference)
import jax
import jax.numpy as jnp
from jax import lax
from jax.experimental import pallas as pl
from jax.experimental.pallas import tpu as pltpu
from jax.experimental.pallas import tpu_sc as plsc

_B = 16384
_N = 200
_D = 64
_TOTAL = _B * _N
_NC = 2
_NS = 16
_NW = _NC * _NS
_PER_W = _TOTAL // _NW
_C = 1024
_CHUNKS = _PER_W // _C


def _gather_body(idx_hbm, table_hbm, out_hbm, idx_v, rows_v, sem):
    wid = lax.axis_index("s") * _NC + lax.axis_index("c")
    base = wid * _PER_W

    def chunk(i, carry):
        off = base + i * _C
        pltpu.sync_copy(idx_hbm.at[pl.ds(off, _C)], idx_v)
        pltpu.async_copy(table_hbm.at[idx_v], rows_v, sem).wait()
        pltpu.sync_copy(rows_v, out_hbm.at[pl.ds(off, _C)])
        return carry

    lax.fori_loop(0, _CHUNKS, chunk, 0)


def kernel(location_ids, table):
    idx = location_ids.reshape(_TOTAL)
    mesh = plsc.VectorSubcoreMesh(core_axis_name="c", subcore_axis_name="s")
    out = pl.kernel(
        _gather_body,
        out_type=jax.ShapeDtypeStruct((_TOTAL, _D), jnp.float32),
        mesh=mesh,
        scratch_types=[
            pltpu.VMEM((_C,), jnp.int32),
            pltpu.VMEM((_C, _D), jnp.float32),
            pltpu.SemaphoreType.DMA,
        ],
        compiler_params=pltpu.CompilerParams(use_tc_tiling_on_sc=False),
    )(idx, table)
    return out.reshape(_B, _N, _D)


# speedup vs baseline: 1.0264x; 1.0264x over previous
import jax
import jax.numpy as jnp
from jax import lax
from jax.experimental import pallas as pl
from jax.experimental.pallas import tpu as pltpu
from jax.experimental.pallas import tpu_sc as plsc

_B = 16384
_N = 200
_D = 64
_TOTAL = _B * _N
_NC = 2
_NS = 16
_NW = _NC * _NS
_PER_W = _TOTAL // _NW
_C = 512
_CHUNKS = _PER_W // _C
_G = _CHUNKS // 2


def _gather_body(idx_hbm, table_hbm, out_hbm,
                 idx0, idx1, rows0, rows1,
                 si0, si1, sg0, sg1, so0, so1):
    wid = lax.axis_index("s") * _NC + lax.axis_index("c")
    base = wid * _PER_W

    idx_v = (idx0, idx1)
    rows_v = (rows0, rows1)
    si = (si0, si1)
    sg = (sg0, sg1)
    so = (so0, so1)

    def idx_cp(i, b):
        return pltpu.make_async_copy(
            idx_hbm.at[pl.ds(base + i * _C, _C)], idx_v[b], si[b])

    def gather_cp(b):
        return pltpu.make_async_copy(table_hbm.at[idx_v[b]], rows_v[b], sg[b])

    def out_cp(i, b):
        return pltpu.make_async_copy(
            rows_v[b], out_hbm.at[pl.ds(base + i * _C, _C)], so[b])

    idx_cp(0, 0).start()
    idx_cp(1, 1).start()
    idx_cp(0, 0).wait()
    gather_cp(0).start()
    gather_cp(0).wait()
    out_cp(0, 0).start()
    idx_cp(2, 0).start()
    idx_cp(1, 1).wait()
    gather_cp(1).start()
    gather_cp(1).wait()
    out_cp(1, 1).start()
    idx_cp(3, 1).start()
    idx_cp(2, 0).wait()
    out_cp(0, 0).wait()
    gather_cp(0).start()

    def body(g, carry):
        i0 = 2 * g
        i1 = i0 + 1
        gather_cp(0).wait()
        out_cp(i0, 0).start()
        idx_cp(i0 + 2, 0).start()
        idx_cp(i1, 1).wait()
        out_cp(i1 - 2, 1).wait()
        gather_cp(1).start()
        gather_cp(1).wait()
        out_cp(i1, 1).start()
        idx_cp(i1 + 2, 1).start()
        idx_cp(i0 + 2, 0).wait()
        out_cp(i0, 0).wait()
        gather_cp(0).start()
        return carry

    lax.fori_loop(1, _G - 1, body, 0)

    iA = _CHUNKS - 2
    iB = _CHUNKS - 1
    gather_cp(0).wait()
    out_cp(iA, 0).start()
    idx_cp(iB, 1).wait()
    out_cp(iB - 2, 1).wait()
    gather_cp(1).start()
    gather_cp(1).wait()
    out_cp(iB, 1).start()
    out_cp(iA, 0).wait()
    out_cp(iB, 1).wait()


def kernel(location_ids, table):
    idx = location_ids.reshape(_TOTAL)
    mesh = plsc.VectorSubcoreMesh(core_axis_name="c", subcore_axis_name="s")
    out = pl.kernel(
        _gather_body,
        out_type=jax.ShapeDtypeStruct((_TOTAL, _D), jnp.float32),
        mesh=mesh,
        scratch_types=[
            pltpu.VMEM((_C,), jnp.int32),
            pltpu.VMEM((_C,), jnp.int32),
            pltpu.VMEM((_C, _D), jnp.float32),
            pltpu.VMEM((_C, _D), jnp.float32),
            pltpu.SemaphoreType.DMA,
            pltpu.SemaphoreType.DMA,
            pltpu.SemaphoreType.DMA,
            pltpu.SemaphoreType.DMA,
            pltpu.SemaphoreType.DMA,
            pltpu.SemaphoreType.DMA,
        ],
        compiler_params=pltpu.CompilerParams(use_tc_tiling_on_sc=False),
    )(idx, table)
    return out.reshape(_B, _N, _D)


# speedup vs baseline: 1.0290x; 1.0026x over previous
import jax
import jax.numpy as jnp
from jax import lax
from jax.experimental import pallas as pl
from jax.experimental.pallas import tpu as pltpu
from jax.experimental.pallas import tpu_sc as plsc

_B = 16384
_N = 200
_D = 64
_TOTAL = _B * _N
_NC = 2
_NS = 16
_NW = _NC * _NS
_PER_W = _TOTAL // _NW
_C = 512
_CHUNKS = _PER_W // _C
_G = _CHUNKS // 2
_SUB = 4
_W = _C // _SUB


def _gather_body(idx_hbm, table_hbm, out_hbm,
                 idx0, idx1, rows0, rows1,
                 si0, si1, sg0, sg1, so0, so1):
    wid = lax.axis_index("s") * _NC + lax.axis_index("c")
    base = wid * _PER_W

    idx_v = (idx0, idx1)
    rows_v = (rows0, rows1)
    si = (si0, si1)
    sg = (sg0, sg1)
    so = (so0, so1)

    def idx_cp(i, b):
        return pltpu.make_async_copy(
            idx_hbm.at[pl.ds(base + i * _C, _C)], idx_v[b], si[b])

    def gather_start(b):
        for j in range(_SUB):
            pltpu.make_async_copy(
                table_hbm.at[idx_v[b].at[pl.ds(j * _W, _W)]],
                rows_v[b].at[pl.ds(j * _W, _W)],
                sg[b]).start()

    def gather_wait(b):
        pltpu.make_async_copy(table_hbm.at[idx_v[b]], rows_v[b], sg[b]).wait()

    def out_cp(i, b):
        return pltpu.make_async_copy(
            rows_v[b], out_hbm.at[pl.ds(base + i * _C, _C)], so[b])

    idx_cp(0, 0).start()
    idx_cp(1, 1).start()
    idx_cp(0, 0).wait()
    gather_start(0)
    gather_wait(0)
    out_cp(0, 0).start()
    idx_cp(2, 0).start()
    idx_cp(1, 1).wait()
    gather_start(1)
    gather_wait(1)
    out_cp(1, 1).start()
    idx_cp(3, 1).start()
    idx_cp(2, 0).wait()
    out_cp(0, 0).wait()
    gather_start(0)

    def body(g, carry):
        i0 = 2 * g
        i1 = i0 + 1
        gather_wait(0)
        out_cp(i0, 0).start()
        idx_cp(i0 + 2, 0).start()
        idx_cp(i1, 1).wait()
        out_cp(i1 - 2, 1).wait()
        gather_start(1)
        gather_wait(1)
        out_cp(i1, 1).start()
        idx_cp(i1 + 2, 1).start()
        idx_cp(i0 + 2, 0).wait()
        out_cp(i0, 0).wait()
        gather_start(0)
        return carry

    lax.fori_loop(1, _G - 1, body, 0)

    iA = _CHUNKS - 2
    iB = _CHUNKS - 1
    gather_wait(0)
    out_cp(iA, 0).start()
    idx_cp(iB, 1).wait()
    out_cp(iB - 2, 1).wait()
    gather_start(1)
    gather_wait(1)
    out_cp(iB, 1).start()
    out_cp(iA, 0).wait()
    out_cp(iB, 1).wait()


def kernel(location_ids, table):
    idx = location_ids.reshape(_TOTAL)
    mesh = plsc.VectorSubcoreMesh(core_axis_name="c", subcore_axis_name="s")
    out = pl.kernel(
        _gather_body,
        out_type=jax.ShapeDtypeStruct((_TOTAL, _D), jnp.float32),
        mesh=mesh,
        scratch_types=[
            pltpu.VMEM((_C,), jnp.int32),
            pltpu.VMEM((_C,), jnp.int32),
            pltpu.VMEM((_C, _D), jnp.float32),
            pltpu.VMEM((_C, _D), jnp.float32),
            pltpu.SemaphoreType.DMA,
            pltpu.SemaphoreType.DMA,
            pltpu.SemaphoreType.DMA,
            pltpu.SemaphoreType.DMA,
            pltpu.SemaphoreType.DMA,
            pltpu.SemaphoreType.DMA,
        ],
        compiler_params=pltpu.CompilerParams(use_tc_tiling_on_sc=False),
    )(idx, table)
    return out.reshape(_B, _N, _D)
